# baseline (speedup 1.0000x reference)
import jax
import jax.numpy as jnp
from jax.experimental import pallas as pl
from jax.experimental.pallas import tpu as pltpu

_N_RF = 8


def _fused_body(xyz_ref, rots_ref, offs_ref, W1_ref, b1_ref, Waw_ref,
                sigma_ref, feat_ref):
    xb = xyz_ref[...].astype(jnp.bfloat16).astype(jnp.float32)
    best_clip = None
    sigma = None
    feat = None
    for r in range(_N_RF):
        rot = rots_ref[r].astype(jnp.float32)
        w1 = W1_ref[r].astype(jnp.float32)
        off = offs_ref[r]
        cols = []
        for i in range(3):
            cols.append(xb[:, 0:1] * rot[i:i + 1, 0:1]
                        + xb[:, 1:2] * rot[i:i + 1, 1:2]
                        + xb[:, 2:3] * rot[i:i + 1, 2:3]
                        + off[0:1, i:i + 1])
        pre = b1_ref[r][None, :]
        for i in range(3):
            ox_i = cols[i].astype(jnp.bfloat16).astype(jnp.float32)
            pre = pre + ox_i * w1[i:i + 1, :]
        h = jnp.maximum(pre, 0.0)
        out = jax.lax.dot_general(
            h.astype(jnp.bfloat16), Waw_ref[r], (((1,), (0,)), ((), ())),
            preferred_element_type=jnp.float32)
        ft = out[:, :128]
        sig = out[:, 128:129]
        clip = jnp.clip(sig, -10.0, 10.0)
        if r == 0:
            best_clip, sigma, feat = clip, sig, ft
        else:
            upd = clip > best_clip
            best_clip = jnp.where(upd, clip, best_clip)
            sigma = jnp.where(upd, sig, sigma)
            feat = jnp.where(upd, ft, feat)
    sigma_ref[...] = sigma
    feat_ref[...] = feat


def kernel(xyz, rots, offsets, aabbs, W1, b1, wd, Wa):
    del aabbs
    n = xyz.shape[0]
    blk = 1024
    grid = (n // blk,)
    Waw = jnp.concatenate(
        [Wa, wd[:, :, None], jnp.zeros((_N_RF, 256, 127), jnp.float32)],
        axis=2).astype(jnp.bfloat16)
    whole = lambda *dims: pl.BlockSpec(dims, lambda i: (0,) * len(dims))
    sigma2, feat = pl.pallas_call(
        _fused_body,
        grid=grid,
        in_specs=[
            pl.BlockSpec((blk, 3), lambda i: (i, 0)),
            whole(_N_RF, 3, 3),
            whole(_N_RF, 1, 3),
            whole(_N_RF, 3, 256),
            whole(_N_RF, 256),
            whole(_N_RF, 256, 256),
        ],
        out_specs=[
            pl.BlockSpec((blk, 1), lambda i: (i, 0)),
            pl.BlockSpec((blk, 128), lambda i: (i, 0)),
        ],
        out_shape=[
            jax.ShapeDtypeStruct((n, 1), jnp.float32),
            jax.ShapeDtypeStruct((n, 128), jnp.float32),
        ],
        compiler_params=pltpu.CompilerParams(
            dimension_semantics=("parallel",)),
    )(xyz, rots.astype(jnp.bfloat16), offsets[:, :1, :3],
      W1.astype(jnp.bfloat16), b1, Waw)
    return sigma2.reshape(-1), feat


# speedup vs baseline: 3.1592x; 3.1592x over previous
import jax
import jax.numpy as jnp
from jax.experimental import pallas as pl
from jax.experimental.pallas import tpu as pltpu

_N_RF = 8


def _fused_body(xyz_ref, rots_ref, offs_ref, W1_ref, b1_ref, Waw_ref,
                sigma_ref, feat_ref):
    xb = xyz_ref[...].astype(jnp.bfloat16)
    best_clip = None
    sigma = None
    feat = None
    for r in range(_N_RF):
        rxyz = jax.lax.dot_general(
            xb, rots_ref[r], (((1,), (1,)), ((), ())),
            preferred_element_type=jnp.float32)
        oxyz = (rxyz + offs_ref[r]).astype(jnp.bfloat16)
        pre = jax.lax.dot_general(
            oxyz, W1_ref[r], (((1,), (0,)), ((), ())),
            preferred_element_type=jnp.float32)
        h = jnp.maximum(pre + b1_ref[r][None, :], 0.0)
        out = jax.lax.dot_general(
            h.astype(jnp.bfloat16), Waw_ref[r], (((1,), (0,)), ((), ())),
            preferred_element_type=jnp.float32)
        ft = out[:, :128]
        sig = out[:, 128:129]
        clip = jnp.clip(sig, -10.0, 10.0)
        if r == 0:
            best_clip, sigma, feat = clip, sig, ft
        else:
            upd = clip > best_clip
            best_clip = jnp.where(upd, clip, best_clip)
            sigma = jnp.where(upd, sig, sigma)
            feat = jnp.where(upd, ft, feat)
    sigma_ref[...] = sigma
    feat_ref[...] = feat


def kernel(xyz, rots, offsets, aabbs, W1, b1, wd, Wa):
    del aabbs
    n = xyz.shape[0]
    blk = 1024
    grid = (n // blk,)
    Waw = jnp.concatenate(
        [Wa, wd[:, :, None], jnp.zeros((_N_RF, 256, 127), jnp.float32)],
        axis=2).astype(jnp.bfloat16)
    whole = lambda *dims: pl.BlockSpec(dims, lambda i: (0,) * len(dims))
    sigma2, feat = pl.pallas_call(
        _fused_body,
        grid=grid,
        in_specs=[
            pl.BlockSpec((blk, 3), lambda i: (i, 0)),
            whole(_N_RF, 3, 3),
            whole(_N_RF, 1, 3),
            whole(_N_RF, 3, 256),
            whole(_N_RF, 256),
            whole(_N_RF, 256, 256),
        ],
        out_specs=[
            pl.BlockSpec((blk, 1), lambda i: (i, 0)),
            pl.BlockSpec((blk, 128), lambda i: (i, 0)),
        ],
        out_shape=[
            jax.ShapeDtypeStruct((n, 1), jnp.float32),
            jax.ShapeDtypeStruct((n, 128), jnp.float32),
        ],
        compiler_params=pltpu.CompilerParams(
            dimension_semantics=("parallel",)),
    )(xyz, rots.astype(jnp.bfloat16), offsets[:, :1, :3],
      W1.astype(jnp.bfloat16), b1, Waw)
    return sigma2.reshape(-1), feat
